# Initial kernel scaffold; baseline (speedup 1.0000x reference)
import jax
import jax.numpy as jnp
from jax.experimental import pallas as pl


def kernel(x, edge_index, W1, b1, W2, b2, W3, b3, W4, b4, fw1, fb1, fw2, fb2, fw3, fb3):
    raise NotImplementedError("write your pallas kernel here")


# speedup vs baseline: 18.6899x; 18.6899x over previous
import functools

import jax
import jax.numpy as jnp
from jax import lax
from jax.experimental import pallas as pl
from jax.experimental.pallas import tpu as pltpu
from jax.experimental.pallas import tpu_sc as plsc

N = 10000
NPAD = 10240
E = 320000
LANES = 128
EROWS = 2560
EPAD = EROWS * LANES
NCORES = 2
NSUB = 16
TROWS = EROWS // NSUB
IB = 16
DROWS = EROWS // (NCORES * NSUB)
STRIPE = NPAD // NSUB
RB = 1024
GRID = NPAD // RB
D_IN = 128
D_H = 256


@functools.cache
def _sc_kernels():
    mesh = plsc.VectorSubcoreMesh(core_axis_name="c", subcore_axis_name="s",
                                  num_cores=NCORES, num_subcores=NSUB)

    deg = functools.partial(
        pl.kernel,
        out_type=jax.ShapeDtypeStruct((NCORES, NPAD), jnp.float32),
        mesh=mesh,
        scratch_types=[
            pltpu.VMEM((DROWS, LANES), jnp.int32),
            pltpu.VMEM((LANES,), jnp.float32),
            pltpu.VMEM((STRIPE,), jnp.float32),
            pltpu.VMEM_SHARED((NPAD,), jnp.float32),
        ],
    )(_sc_degree_body)

    agg = functools.partial(
        pl.kernel,
        out_type=jax.ShapeDtypeStruct((NCORES, NPAD, LANES), jnp.float32),
        mesh=mesh,
        scratch_types=[
            pltpu.VMEM((IB, LANES), jnp.int32),
            pltpu.VMEM((IB, LANES), jnp.int32),
            pltpu.VMEM((LANES, LANES), jnp.float32),
            pltpu.VMEM((LANES, LANES), jnp.float32),
            pltpu.VMEM_SHARED((NPAD, LANES), jnp.float32),
            pltpu.SemaphoreType.DMA,
            pltpu.SemaphoreType.DMA,
        ],
    )(_sc_aggregate_body)

    return deg, agg


def _sc_degree_body(dst_hbm, out_hbm, dstb, ones, stage, acc):
    c = lax.axis_index("c")
    s = lax.axis_index("s")
    w = s * NCORES + c

    for j in range(LANES // 16):
        ones[pl.ds(j * 16, 16)] = jnp.ones((16,), jnp.float32)

    iv = jnp.where(c == 0, 1.0, 0.0).astype(jnp.float32)

    def _fill(i, carry):
        stage[pl.ds(i * 16, 16)] = jnp.broadcast_to(iv, (16,))
        return carry

    lax.fori_loop(0, STRIPE // 16, _fill, 0)
    pltpu.sync_copy(stage, acc.at[pl.ds(s * STRIPE, STRIPE)])
    pltpu.sync_copy(dst_hbm.at[pl.ds(w * DROWS, DROWS)], dstb)
    plsc.subcore_barrier()

    def _row(i, carry):
        pltpu.sync_copy(ones, acc.at[dstb.at[i]], add=True)
        return carry

    lax.fori_loop(0, DROWS, _row, 0)
    plsc.subcore_barrier()

    pltpu.sync_copy(acc.at[pl.ds(s * STRIPE, STRIPE)], stage)
    pltpu.sync_copy(stage, out_hbm.at[c, pl.ds(s * STRIPE, STRIPE)])


def _sc_aggregate_body(g_hbm, src_hbm, dst_hbm, out_hbm,
                       srcb, dstb, buf0, buf1, acc, sem0, sem1):
    c = lax.axis_index("c")
    s = lax.axis_index("s")

    def _zrow(i, carry):
        for j in range(LANES // 16):
            buf0[i, pl.ds(j * 16, 16)] = jnp.zeros((16,), jnp.float32)
        return carry

    lax.fori_loop(0, LANES, _zrow, 0)
    for k in range(STRIPE // LANES):
        pltpu.sync_copy(buf0, acc.at[pl.ds(s * STRIPE + k * LANES, LANES)])
    plsc.subcore_barrier()

    def _blk(bi, carry):
        base = s * TROWS + bi * IB
        pltpu.sync_copy(src_hbm.at[c, pl.ds(base, IB)], srcb)
        pltpu.sync_copy(dst_hbm.at[pl.ds(base, IB)], dstb)
        pltpu.async_copy(g_hbm.at[srcb.at[0]], buf0, sem0)

        def _pair(t, carry2):
            r0 = 2 * t
            r1 = r0 + 1
            d1 = pltpu.async_copy(g_hbm.at[srcb.at[r1]], buf1, sem1)
            pltpu.make_async_copy(g_hbm.at[srcb.at[r0]], buf0, sem0).wait()
            pltpu.sync_copy(buf0, acc.at[dstb.at[r0]], add=True)

            @pl.when(t < IB // 2 - 1)
            def _():
                pltpu.async_copy(g_hbm.at[srcb.at[r0 + 2]], buf0, sem0)

            d1.wait()
            pltpu.sync_copy(buf1, acc.at[dstb.at[r1]], add=True)
            return carry2

        lax.fori_loop(0, IB // 2, _pair, 0)
        return carry

    lax.fori_loop(0, TROWS // IB, _blk, 0)
    plsc.subcore_barrier()

    for k in range(STRIPE // LANES):
        base = s * STRIPE + k * LANES
        pltpu.sync_copy(acc.at[pl.ds(base, LANES)], buf0)
        pltpu.sync_copy(buf0, out_hbm.at[c, pl.ds(base, LANES)])


def _dinv_col(deg_ref):
    d = deg_ref[0] + deg_ref[1]
    dv = lax.rsqrt(d)
    t = dv.T
    return jnp.concatenate([t[:, k:k + 1] for k in range(8)], axis=0)


def _tc_first_body(deg_ref, x_ref, w_ref, g_ref, dinv_ref):
    col = _dinv_col(deg_ref)
    h = jnp.dot(x_ref[...], w_ref[...], preferred_element_type=jnp.float32)
    g = h * col
    g_ref[0] = g[:, :LANES]
    g_ref[1] = g[:, LANES:]
    dinv_ref[...] = jnp.broadcast_to(col, (RB, LANES))


def _tc_mid_body(s_ref, gp_ref, dinv_ref, b_ref, w_ref, g_ref):
    dv = dinv_ref[...]
    zl = jnp.maximum((s_ref[0] + gp_ref[0]) * dv + b_ref[0:1, :LANES], 0.0)
    zr = jnp.maximum((s_ref[1] + gp_ref[1]) * dv + b_ref[0:1, LANES:], 0.0)
    h = (jnp.dot(zl, w_ref[:LANES, :], preferred_element_type=jnp.float32)
         + jnp.dot(zr, w_ref[LANES:, :], preferred_element_type=jnp.float32))
    g = h * dv[:, 0:1]
    g_ref[0] = g[:, :LANES]
    g_ref[1] = g[:, LANES:]


def _tc_final_body(s_ref, gp_ref, dinv_ref, b_ref,
                   fw1_ref, fb1_ref, fw2_ref, fb2_ref, fw3_ref, fb3_ref,
                   o_ref):
    dv = dinv_ref[...]
    zl = jnp.maximum((s_ref[0] + gp_ref[0]) * dv + b_ref[0:1, :LANES], 0.0)
    zr = jnp.maximum((s_ref[1] + gp_ref[1]) * dv + b_ref[0:1, LANES:], 0.0)
    z = jnp.concatenate([zl, zr], axis=1)
    h1 = jnp.maximum(
        jnp.dot(z, fw1_ref[...], preferred_element_type=jnp.float32)
        + fb1_ref[0:1, :], 0.0)
    h2 = jnp.maximum(
        jnp.dot(h1, fw2_ref[...], preferred_element_type=jnp.float32)
        + fb2_ref[0:1, :], 0.0)
    o_ref[...] = (jnp.dot(h2, fw3_ref[...], preferred_element_type=jnp.float32)
                  + fb3_ref[0:1, :])


_f32 = jnp.float32
_gspec = pl.BlockSpec((NCORES, RB, LANES), lambda i: (0, i, 0))
_nspec = pl.BlockSpec((RB, LANES), lambda i: (i, 0))
_gshape = jax.ShapeDtypeStruct((NCORES, NPAD, LANES), _f32)

_tc_first_specs = [pl.BlockSpec((NCORES, 8, LANES), lambda i: (0, i, 0)),
                   _nspec,
                   pl.BlockSpec((D_IN, D_H), lambda i: (0, 0))]
_tc_first_outspecs = [_gspec, _nspec]
_tc_first_outshape = [_gshape, jax.ShapeDtypeStruct((NPAD, LANES), _f32)]
_tc_mid_specs = [_gspec, _gspec, _nspec,
                 pl.BlockSpec((8, D_H), lambda i: (0, 0)),
                 pl.BlockSpec((D_H, D_H), lambda i: (0, 0))]
_tc_final_specs = [_gspec, _gspec, _nspec,
                   pl.BlockSpec((8, D_H), lambda i: (0, 0)),
                   pl.BlockSpec((D_H, D_H), lambda i: (0, 0)),
                   pl.BlockSpec((8, D_H), lambda i: (0, 0)),
                   pl.BlockSpec((D_H, LANES), lambda i: (0, 0)),
                   pl.BlockSpec((8, LANES), lambda i: (0, 0)),
                   pl.BlockSpec((LANES, LANES), lambda i: (0, 0)),
                   pl.BlockSpec((8, LANES), lambda i: (0, 0))]
_tc_final_outshape = jax.ShapeDtypeStruct((NPAD, LANES), _f32)

_tc_first = pl.pallas_call(
    _tc_first_body, grid=(GRID,), in_specs=_tc_first_specs,
    out_specs=_tc_first_outspecs, out_shape=_tc_first_outshape)

_tc_mid = pl.pallas_call(
    _tc_mid_body, grid=(GRID,), in_specs=_tc_mid_specs,
    out_specs=_gspec, out_shape=_gshape)

_tc_final = pl.pallas_call(
    _tc_final_body, grid=(GRID,), in_specs=_tc_final_specs,
    out_specs=_nspec, out_shape=_tc_final_outshape)


def _bcast8(b):
    return jnp.broadcast_to(b[None, :], (8, b.shape[0]))


def kernel(x, edge_index, W1, b1, W2, b2, W3, b3, W4, b4,
           fw1, fb1, fw2, fb2, fw3, fb3):
    src = edge_index[0].astype(jnp.int32)
    dst = edge_index[1].astype(jnp.int32)
    pad = EPAD - E
    pi = jnp.arange(pad, dtype=jnp.int32)
    srcp = jnp.concatenate([src, pi % N])
    dstp = jnp.concatenate([dst, N + pi % (NPAD - N)])
    src2 = jnp.stack([srcp, srcp + NPAD]).reshape(NCORES, EROWS, LANES)
    dst2 = dstp.reshape(EROWS, LANES)

    sc_degree, sc_aggregate = _sc_kernels()
    deg = sc_degree(dst2)
    deg2d = deg.reshape(NCORES, NPAD // LANES, LANES)
    xp = jnp.pad(x.astype(_f32), ((0, NPAD - N), (0, 0)))

    g, dinv = _tc_first(deg2d, xp, W1)
    for b, W in ((b1, W2), (b2, W3), (b3, W4)):
        sagg = sc_aggregate(g.reshape(NCORES * NPAD, LANES), src2, dst2)
        g = _tc_mid(sagg, g, dinv, _bcast8(b), W)
    sagg = sc_aggregate(g.reshape(NCORES * NPAD, LANES), src2, dst2)
    out = _tc_final(sagg, g, dinv, _bcast8(b4),
                    fw1, _bcast8(fb1), fw2, _bcast8(fb2), fw3, _bcast8(fb3))
    return out[:N]


# speedup vs baseline: 18.9234x; 1.0125x over previous
import functools

import jax
import jax.numpy as jnp
from jax import lax
from jax.experimental import pallas as pl
from jax.experimental.pallas import tpu as pltpu
from jax.experimental.pallas import tpu_sc as plsc

N = 10000
NPAD = 10240
E = 320000
LANES = 128
EROWS = 2560
EPAD = EROWS * LANES
NCORES = 2
NSUB = 16
CH = 64
CHROWS = EPAD // CH
TCH = CHROWS // NSUB
IB = 32
NBLK = TCH // IB
DROWS = CHROWS // (NCORES * NSUB)
STRIPE = NPAD // NSUB
RB = 1024
GRID = NPAD // RB
D_IN = 128
D_H = 256


@functools.cache
def _sc_kernels():
    mesh = plsc.VectorSubcoreMesh(core_axis_name="c", subcore_axis_name="s",
                                  num_cores=NCORES, num_subcores=NSUB)

    deg = functools.partial(
        pl.kernel,
        out_type=jax.ShapeDtypeStruct((NCORES, NPAD), jnp.float32),
        mesh=mesh,
        scratch_types=[
            pltpu.VMEM((DROWS, CH), jnp.int32),
            pltpu.VMEM((CH,), jnp.float32),
            pltpu.VMEM((STRIPE,), jnp.float32),
            pltpu.VMEM_SHARED((NPAD,), jnp.float32),
        ],
    )(_sc_degree_body)

    agg = functools.partial(
        pl.kernel,
        out_type=jax.ShapeDtypeStruct((NCORES, NPAD, LANES), jnp.float32),
        mesh=mesh,
        scratch_types=[
            pltpu.VMEM((IB, CH), jnp.int32),
            pltpu.VMEM((IB, CH), jnp.int32),
            pltpu.VMEM((CH, LANES), jnp.float32),
            pltpu.VMEM((CH, LANES), jnp.float32),
            pltpu.VMEM((CH, LANES), jnp.float32),
            pltpu.VMEM((CH, LANES), jnp.float32),
            pltpu.VMEM_SHARED((NPAD, LANES), jnp.float32),
            pltpu.SemaphoreType.DMA, pltpu.SemaphoreType.DMA,
            pltpu.SemaphoreType.DMA, pltpu.SemaphoreType.DMA,
            pltpu.SemaphoreType.DMA, pltpu.SemaphoreType.DMA,
            pltpu.SemaphoreType.DMA, pltpu.SemaphoreType.DMA,
        ],
    )(_sc_aggregate_body)

    return deg, agg


def _sc_degree_body(dst_hbm, out_hbm, dstb, ones, stage, acc):
    c = lax.axis_index("c")
    s = lax.axis_index("s")
    w = s * NCORES + c

    for j in range(CH // 16):
        ones[pl.ds(j * 16, 16)] = jnp.ones((16,), jnp.float32)

    iv = jnp.where(c == 0, 1.0, 0.0).astype(jnp.float32)

    def _fill(i, carry):
        stage[pl.ds(i * 16, 16)] = jnp.broadcast_to(iv, (16,))
        return carry

    lax.fori_loop(0, STRIPE // 16, _fill, 0)
    pltpu.sync_copy(stage, acc.at[pl.ds(s * STRIPE, STRIPE)])
    pltpu.sync_copy(dst_hbm.at[pl.ds(w * DROWS, DROWS)], dstb)
    plsc.subcore_barrier()

    def _row(i, carry):
        pltpu.sync_copy(ones, acc.at[dstb.at[i]], add=True)
        return carry

    lax.fori_loop(0, DROWS, _row, 0)
    plsc.subcore_barrier()

    pltpu.sync_copy(acc.at[pl.ds(s * STRIPE, STRIPE)], stage)
    pltpu.sync_copy(stage, out_hbm.at[c, pl.ds(s * STRIPE, STRIPE)])


def _sc_aggregate_body(g_hbm, src_hbm, dst_hbm, out_hbm,
                       srcb, dstb, b0, b1, b2, b3, acc,
                       sg0, sg1, sg2, sg3, ss0, ss1, ss2, ss3):
    c = lax.axis_index("c")
    s = lax.axis_index("s")
    bufs = (b0, b1, b2, b3)
    sgs = (sg0, sg1, sg2, sg3)
    sss = (ss0, ss1, ss2, ss3)

    def _zrow(i, carry):
        for j in range(LANES // 16):
            b0[i, pl.ds(j * 16, 16)] = jnp.zeros((16,), jnp.float32)
        return carry

    lax.fori_loop(0, CH, _zrow, 0)
    for k in range(STRIPE // CH):
        pltpu.sync_copy(b0, acc.at[pl.ds(s * STRIPE + k * CH, CH)])
    plsc.subcore_barrier()

    def _blk(bi, carry):
        base = s * TCH + bi * IB
        pltpu.sync_copy(src_hbm.at[c, pl.ds(base, IB)], srcb)
        pltpu.sync_copy(dst_hbm.at[pl.ds(base, IB)], dstb)
        pltpu.async_copy(g_hbm.at[srcb.at[0]], b0, sg0)
        pltpu.async_copy(g_hbm.at[srcb.at[1]], b1, sg1)

        def _quad(q, carry2):
            for u in range(4):
                j = 4 * q + u
                v = (u + 2) % 4
                if u < 2:
                    @pl.when(q >= 1)
                    def _():
                        pltpu.make_async_copy(
                            bufs[v], acc.at[dstb.at[j - 2]], sss[v]).wait()
                    pltpu.async_copy(g_hbm.at[srcb.at[j + 2]], bufs[v], sgs[v])
                else:
                    pltpu.make_async_copy(
                        bufs[v], acc.at[dstb.at[j - 2]], sss[v]).wait()

                    @pl.when(q < IB // 4 - 1)
                    def _():
                        pltpu.async_copy(g_hbm.at[srcb.at[j + 2]],
                                         bufs[v], sgs[v])
                pltpu.make_async_copy(g_hbm.at[srcb.at[j]],
                                      bufs[u], sgs[u]).wait()
                pltpu.async_copy(bufs[u], acc.at[dstb.at[j]], sss[u], add=True)
            return carry2

        lax.fori_loop(0, IB // 4, _quad, 0)
        pltpu.make_async_copy(b2, acc.at[dstb.at[IB - 2]], ss2).wait()
        pltpu.make_async_copy(b3, acc.at[dstb.at[IB - 1]], ss3).wait()
        return carry

    lax.fori_loop(0, NBLK, _blk, 0)
    plsc.subcore_barrier()

    for k in range(STRIPE // CH):
        base = s * STRIPE + k * CH
        pltpu.sync_copy(acc.at[pl.ds(base, CH)], b0)
        pltpu.sync_copy(b0, out_hbm.at[c, pl.ds(base, CH)])


def _dinv_col(deg_ref):
    d = deg_ref[0] + deg_ref[1]
    dv = lax.rsqrt(d)
    t = dv.T
    return jnp.concatenate([t[:, k:k + 1] for k in range(8)], axis=0)


def _tc_first_body(deg_ref, x_ref, w_ref, g_ref, dinv_ref):
    col = _dinv_col(deg_ref)
    h = jnp.dot(x_ref[...], w_ref[...], preferred_element_type=jnp.float32)
    g = h * col
    g_ref[0] = g[:, :LANES]
    g_ref[1] = g[:, LANES:]
    dinv_ref[...] = jnp.broadcast_to(col, (RB, LANES))


def _tc_mid_body(s_ref, gp_ref, dinv_ref, b_ref, w_ref, g_ref):
    dv = dinv_ref[...]
    zl = jnp.maximum((s_ref[0] + gp_ref[0]) * dv + b_ref[0:1, :LANES], 0.0)
    zr = jnp.maximum((s_ref[1] + gp_ref[1]) * dv + b_ref[0:1, LANES:], 0.0)
    h = (jnp.dot(zl, w_ref[:LANES, :], preferred_element_type=jnp.float32)
         + jnp.dot(zr, w_ref[LANES:, :], preferred_element_type=jnp.float32))
    g = h * dv[:, 0:1]
    g_ref[0] = g[:, :LANES]
    g_ref[1] = g[:, LANES:]


def _tc_final_body(s_ref, gp_ref, dinv_ref, b_ref,
                   fw1_ref, fb1_ref, fw2_ref, fb2_ref, fw3_ref, fb3_ref,
                   o_ref):
    dv = dinv_ref[...]
    zl = jnp.maximum((s_ref[0] + gp_ref[0]) * dv + b_ref[0:1, :LANES], 0.0)
    zr = jnp.maximum((s_ref[1] + gp_ref[1]) * dv + b_ref[0:1, LANES:], 0.0)
    z = jnp.concatenate([zl, zr], axis=1)
    h1 = jnp.maximum(
        jnp.dot(z, fw1_ref[...], preferred_element_type=jnp.float32)
        + fb1_ref[0:1, :], 0.0)
    h2 = jnp.maximum(
        jnp.dot(h1, fw2_ref[...], preferred_element_type=jnp.float32)
        + fb2_ref[0:1, :], 0.0)
    o_ref[...] = (jnp.dot(h2, fw3_ref[...], preferred_element_type=jnp.float32)
                  + fb3_ref[0:1, :])


_f32 = jnp.float32
_gspec = pl.BlockSpec((NCORES, RB, LANES), lambda i: (0, i, 0))
_nspec = pl.BlockSpec((RB, LANES), lambda i: (i, 0))
_gshape = jax.ShapeDtypeStruct((NCORES, NPAD, LANES), _f32)

_tc_first_specs = [pl.BlockSpec((NCORES, 8, LANES), lambda i: (0, i, 0)),
                   _nspec,
                   pl.BlockSpec((D_IN, D_H), lambda i: (0, 0))]
_tc_first_outspecs = [_gspec, _nspec]
_tc_first_outshape = [_gshape, jax.ShapeDtypeStruct((NPAD, LANES), _f32)]
_tc_mid_specs = [_gspec, _gspec, _nspec,
                 pl.BlockSpec((8, D_H), lambda i: (0, 0)),
                 pl.BlockSpec((D_H, D_H), lambda i: (0, 0))]
_tc_final_specs = [_gspec, _gspec, _nspec,
                   pl.BlockSpec((8, D_H), lambda i: (0, 0)),
                   pl.BlockSpec((D_H, D_H), lambda i: (0, 0)),
                   pl.BlockSpec((8, D_H), lambda i: (0, 0)),
                   pl.BlockSpec((D_H, LANES), lambda i: (0, 0)),
                   pl.BlockSpec((8, LANES), lambda i: (0, 0)),
                   pl.BlockSpec((LANES, LANES), lambda i: (0, 0)),
                   pl.BlockSpec((8, LANES), lambda i: (0, 0))]
_tc_final_outshape = jax.ShapeDtypeStruct((NPAD, LANES), _f32)

_tc_first = pl.pallas_call(
    _tc_first_body, grid=(GRID,), in_specs=_tc_first_specs,
    out_specs=_tc_first_outspecs, out_shape=_tc_first_outshape)

_tc_mid = pl.pallas_call(
    _tc_mid_body, grid=(GRID,), in_specs=_tc_mid_specs,
    out_specs=_gspec, out_shape=_gshape)

_tc_final = pl.pallas_call(
    _tc_final_body, grid=(GRID,), in_specs=_tc_final_specs,
    out_specs=_nspec, out_shape=_tc_final_outshape)


def _bcast8(b):
    return jnp.broadcast_to(b[None, :], (8, b.shape[0]))


def kernel(x, edge_index, W1, b1, W2, b2, W3, b3, W4, b4,
           fw1, fb1, fw2, fb2, fw3, fb3):
    src = edge_index[0].astype(jnp.int32)
    dst = edge_index[1].astype(jnp.int32)
    pad = EPAD - E
    pi = jnp.arange(pad, dtype=jnp.int32)
    srcp = jnp.concatenate([src, pi % N])
    dstp = jnp.concatenate([dst, N + pi % (NPAD - N)])
    src2 = jnp.stack([srcp, srcp + NPAD]).reshape(NCORES, CHROWS, CH)
    dst2 = dstp.reshape(CHROWS, CH)

    sc_degree, sc_aggregate = _sc_kernels()
    deg = sc_degree(dst2)
    deg2d = deg.reshape(NCORES, NPAD // LANES, LANES)
    xp = jnp.pad(x.astype(_f32), ((0, NPAD - N), (0, 0)))

    g, dinv = _tc_first(deg2d, xp, W1)
    for b, W in ((b1, W2), (b2, W3), (b3, W4)):
        sagg = sc_aggregate(g.reshape(NCORES * NPAD, LANES), src2, dst2)
        g = _tc_mid(sagg, g, dinv, _bcast8(b), W)
    sagg = sc_aggregate(g.reshape(NCORES * NPAD, LANES), src2, dst2)
    out = _tc_final(sagg, g, dinv, _bcast8(b4),
                    fw1, _bcast8(fb1), fw2, _bcast8(fb2), fw3, _bcast8(fb3))
    return out[:N]


# speedup vs baseline: 19.7754x; 1.0450x over previous
import functools

import jax
import jax.numpy as jnp
from jax import lax
from jax.experimental import pallas as pl
from jax.experimental.pallas import tpu as pltpu
from jax.experimental.pallas import tpu_sc as plsc

N = 10000
NPAD = 10240
E = 320000
LANES = 128
EROWS = 2560
EPAD = EROWS * LANES
NCORES = 2
NSUB = 16
CH = 64
CHROWS = EPAD // CH
TCH = CHROWS // NSUB
IB = 32
NBLK = TCH // IB
DROWS = CHROWS // (NCORES * NSUB)
STRIPE = NPAD // NSUB
RB = 1024
GRID = NPAD // RB
D_IN = 128
D_H = 256


@functools.cache
def _sc_kernels():
    mesh = plsc.VectorSubcoreMesh(core_axis_name="c", subcore_axis_name="s",
                                  num_cores=NCORES, num_subcores=NSUB)

    deg = functools.partial(
        pl.kernel,
        out_type=jax.ShapeDtypeStruct((NCORES, NPAD), jnp.float32),
        mesh=mesh,
        scratch_types=[
            pltpu.VMEM((DROWS, CH), jnp.int32),
            pltpu.VMEM((CH,), jnp.float32),
            pltpu.VMEM((STRIPE,), jnp.float32),
            pltpu.VMEM_SHARED((NPAD,), jnp.float32),
        ],
    )(_sc_degree_body)

    agg = functools.partial(
        pl.kernel,
        out_type=jax.ShapeDtypeStruct((NCORES, NPAD, LANES), jnp.float32),
        mesh=mesh,
        scratch_types=[
            pltpu.VMEM((IB, CH), jnp.int32),
            pltpu.VMEM((IB, CH), jnp.int32),
            pltpu.VMEM((IB, CH), jnp.int32),
            pltpu.VMEM((IB, CH), jnp.int32),
            pltpu.VMEM((CH, LANES), jnp.float32),
            pltpu.VMEM((CH, LANES), jnp.float32),
            pltpu.VMEM((CH, LANES), jnp.float32),
            pltpu.VMEM((CH, LANES), jnp.float32),
            pltpu.VMEM_SHARED((NPAD, LANES), jnp.float32),
            pltpu.SemaphoreType.DMA, pltpu.SemaphoreType.DMA,
            pltpu.SemaphoreType.DMA, pltpu.SemaphoreType.DMA,
            pltpu.SemaphoreType.DMA, pltpu.SemaphoreType.DMA,
            pltpu.SemaphoreType.DMA, pltpu.SemaphoreType.DMA,
            pltpu.SemaphoreType.DMA, pltpu.SemaphoreType.DMA,
        ],
    )(_sc_aggregate_body)

    return deg, agg


def _sc_degree_body(dst_hbm, out_hbm, dstb, ones, stage, acc):
    c = lax.axis_index("c")
    s = lax.axis_index("s")
    w = s * NCORES + c

    for j in range(CH // 16):
        ones[pl.ds(j * 16, 16)] = jnp.ones((16,), jnp.float32)

    iv = jnp.where(c == 0, 1.0, 0.0).astype(jnp.float32)

    def _fill(i, carry):
        stage[pl.ds(i * 16, 16)] = jnp.broadcast_to(iv, (16,))
        return carry

    lax.fori_loop(0, STRIPE // 16, _fill, 0)
    pltpu.sync_copy(stage, acc.at[pl.ds(s * STRIPE, STRIPE)])
    pltpu.sync_copy(dst_hbm.at[pl.ds(w * DROWS, DROWS)], dstb)
    plsc.subcore_barrier()

    def _row(i, carry):
        pltpu.sync_copy(ones, acc.at[dstb.at[i]], add=True)
        return carry

    lax.fori_loop(0, DROWS, _row, 0)
    plsc.subcore_barrier()

    pltpu.sync_copy(acc.at[pl.ds(s * STRIPE, STRIPE)], stage)
    pltpu.sync_copy(stage, out_hbm.at[c, pl.ds(s * STRIPE, STRIPE)])


def _sc_aggregate_body(g_hbm, src_hbm, dst_hbm, out_hbm,
                       srcb0, dstb0, srcb1, dstb1, b0, b1, b2, b3, acc,
                       sg0, sg1, sg2, sg3, ss0, ss1, ss2, ss3, si0, si1):
    c = lax.axis_index("c")
    s = lax.axis_index("s")
    bufs = (b0, b1, b2, b3)
    sgs = (sg0, sg1, sg2, sg3)
    sss = (ss0, ss1, ss2, ss3)

    def _zrow(i, carry):
        for j in range(LANES // 16):
            b0[i, pl.ds(j * 16, 16)] = jnp.zeros((16,), jnp.float32)
        return carry

    lax.fori_loop(0, CH, _zrow, 0)
    for k in range(STRIPE // CH):
        pltpu.sync_copy(b0, acc.at[pl.ds(s * STRIPE + k * CH, CH)])
    plsc.subcore_barrier()

    def _load_idx(base, sb, db, sem):
        pltpu.async_copy(src_hbm.at[c, pl.ds(base, IB)], sb, sem)
        pltpu.async_copy(dst_hbm.at[pl.ds(base, IB)], db, sem)

    def _wait_idx(base, sb, db, sem):
        pltpu.make_async_copy(src_hbm.at[c, pl.ds(base, IB)], sb, sem).wait()
        pltpu.make_async_copy(dst_hbm.at[pl.ds(base, IB)], db, sem).wait()

    def _process(srcb, dstb):
        pltpu.async_copy(g_hbm.at[srcb.at[0]], b0, sg0)
        pltpu.async_copy(g_hbm.at[srcb.at[1]], b1, sg1)

        def _quad(q, carry2):
            for u in range(4):
                j = 4 * q + u
                v = (u + 2) % 4
                if u < 2:
                    @pl.when(q >= 1)
                    def _():
                        pltpu.make_async_copy(
                            bufs[v], acc.at[dstb.at[j - 2]], sss[v]).wait()
                    pltpu.async_copy(g_hbm.at[srcb.at[j + 2]], bufs[v], sgs[v])
                else:
                    pltpu.make_async_copy(
                        bufs[v], acc.at[dstb.at[j - 2]], sss[v]).wait()

                    @pl.when(q < IB // 4 - 1)
                    def _():
                        pltpu.async_copy(g_hbm.at[srcb.at[j + 2]],
                                         bufs[v], sgs[v])
                pltpu.make_async_copy(g_hbm.at[srcb.at[j]],
                                      bufs[u], sgs[u]).wait()
                pltpu.async_copy(bufs[u], acc.at[dstb.at[j]], sss[u], add=True)
            return carry2

        lax.fori_loop(0, IB // 4, _quad, 0)
        pltpu.make_async_copy(b2, acc.at[dstb.at[IB - 2]], ss2).wait()
        pltpu.make_async_copy(b3, acc.at[dstb.at[IB - 1]], ss3).wait()

    tbase = s * TCH
    _load_idx(tbase, srcb0, dstb0, si0)

    def _bb(t, carry):
        base0 = tbase + 2 * t * IB
        _wait_idx(base0, srcb0, dstb0, si0)
        _load_idx(base0 + IB, srcb1, dstb1, si1)
        _process(srcb0, dstb0)
        _wait_idx(base0 + IB, srcb1, dstb1, si1)

        @pl.when(t < NBLK // 2 - 1)
        def _():
            _load_idx(base0 + 2 * IB, srcb0, dstb0, si0)

        _process(srcb1, dstb1)
        return carry

    lax.fori_loop(0, NBLK // 2, _bb, 0)
    plsc.subcore_barrier()

    for k in range(STRIPE // CH):
        base = s * STRIPE + k * CH
        pltpu.sync_copy(acc.at[pl.ds(base, CH)], b0)
        pltpu.sync_copy(b0, out_hbm.at[c, pl.ds(base, CH)])


def _dinv_col(deg_ref):
    d = deg_ref[0] + deg_ref[1]
    dv = lax.rsqrt(d)
    t = dv.T
    return jnp.concatenate([t[:, k:k + 1] for k in range(8)], axis=0)


def _tc_first_body(deg_ref, x_ref, w_ref, g_ref, dinv_ref):
    col = _dinv_col(deg_ref)
    h = jnp.dot(x_ref[...], w_ref[...], preferred_element_type=jnp.float32)
    g = h * col
    g_ref[0] = g[:, :LANES]
    g_ref[1] = g[:, LANES:]
    dinv_ref[...] = jnp.broadcast_to(col, (RB, LANES))


def _tc_mid_body(s_ref, gp_ref, dinv_ref, b_ref, w_ref, g_ref):
    dv = dinv_ref[...]
    zl = jnp.maximum((s_ref[0] + gp_ref[0]) * dv + b_ref[0:1, :LANES], 0.0)
    zr = jnp.maximum((s_ref[1] + gp_ref[1]) * dv + b_ref[0:1, LANES:], 0.0)
    h = (jnp.dot(zl, w_ref[:LANES, :], preferred_element_type=jnp.float32)
         + jnp.dot(zr, w_ref[LANES:, :], preferred_element_type=jnp.float32))
    g = h * dv[:, 0:1]
    g_ref[0] = g[:, :LANES]
    g_ref[1] = g[:, LANES:]


def _tc_final_body(s_ref, gp_ref, dinv_ref, b_ref,
                   fw1_ref, fb1_ref, fw2_ref, fb2_ref, fw3_ref, fb3_ref,
                   o_ref):
    dv = dinv_ref[...]
    zl = jnp.maximum((s_ref[0] + gp_ref[0]) * dv + b_ref[0:1, :LANES], 0.0)
    zr = jnp.maximum((s_ref[1] + gp_ref[1]) * dv + b_ref[0:1, LANES:], 0.0)
    z = jnp.concatenate([zl, zr], axis=1)
    h1 = jnp.maximum(
        jnp.dot(z, fw1_ref[...], preferred_element_type=jnp.float32)
        + fb1_ref[0:1, :], 0.0)
    h2 = jnp.maximum(
        jnp.dot(h1, fw2_ref[...], preferred_element_type=jnp.float32)
        + fb2_ref[0:1, :], 0.0)
    o_ref[...] = (jnp.dot(h2, fw3_ref[...], preferred_element_type=jnp.float32)
                  + fb3_ref[0:1, :])


_f32 = jnp.float32
_gspec = pl.BlockSpec((NCORES, RB, LANES), lambda i: (0, i, 0))
_nspec = pl.BlockSpec((RB, LANES), lambda i: (i, 0))
_gshape = jax.ShapeDtypeStruct((NCORES, NPAD, LANES), _f32)

_tc_first_specs = [pl.BlockSpec((NCORES, 8, LANES), lambda i: (0, i, 0)),
                   _nspec,
                   pl.BlockSpec((D_IN, D_H), lambda i: (0, 0))]
_tc_first_outspecs = [_gspec, _nspec]
_tc_first_outshape = [_gshape, jax.ShapeDtypeStruct((NPAD, LANES), _f32)]
_tc_mid_specs = [_gspec, _gspec, _nspec,
                 pl.BlockSpec((8, D_H), lambda i: (0, 0)),
                 pl.BlockSpec((D_H, D_H), lambda i: (0, 0))]
_tc_final_specs = [_gspec, _gspec, _nspec,
                   pl.BlockSpec((8, D_H), lambda i: (0, 0)),
                   pl.BlockSpec((D_H, D_H), lambda i: (0, 0)),
                   pl.BlockSpec((8, D_H), lambda i: (0, 0)),
                   pl.BlockSpec((D_H, LANES), lambda i: (0, 0)),
                   pl.BlockSpec((8, LANES), lambda i: (0, 0)),
                   pl.BlockSpec((LANES, LANES), lambda i: (0, 0)),
                   pl.BlockSpec((8, LANES), lambda i: (0, 0))]
_tc_final_outshape = jax.ShapeDtypeStruct((NPAD, LANES), _f32)

_tc_first = pl.pallas_call(
    _tc_first_body, grid=(GRID,), in_specs=_tc_first_specs,
    out_specs=_tc_first_outspecs, out_shape=_tc_first_outshape)

_tc_mid = pl.pallas_call(
    _tc_mid_body, grid=(GRID,), in_specs=_tc_mid_specs,
    out_specs=_gspec, out_shape=_gshape)

_tc_final = pl.pallas_call(
    _tc_final_body, grid=(GRID,), in_specs=_tc_final_specs,
    out_specs=_nspec, out_shape=_tc_final_outshape)


def _bcast8(b):
    return jnp.broadcast_to(b[None, :], (8, b.shape[0]))


def kernel(x, edge_index, W1, b1, W2, b2, W3, b3, W4, b4,
           fw1, fb1, fw2, fb2, fw3, fb3):
    src = edge_index[0].astype(jnp.int32)
    dst = edge_index[1].astype(jnp.int32)
    pad = EPAD - E
    pi = jnp.arange(pad, dtype=jnp.int32)
    srcp = jnp.concatenate([src, pi % N])
    dstp = jnp.concatenate([dst, N + pi % (NPAD - N)])
    src2 = jnp.stack([srcp, srcp + NPAD]).reshape(NCORES, CHROWS, CH)
    dst2 = dstp.reshape(CHROWS, CH)

    sc_degree, sc_aggregate = _sc_kernels()
    deg = sc_degree(dst2)
    deg2d = deg.reshape(NCORES, NPAD // LANES, LANES)
    xp = jnp.pad(x.astype(_f32), ((0, NPAD - N), (0, 0)))

    g, dinv = _tc_first(deg2d, xp, W1)
    for b, W in ((b1, W2), (b2, W3), (b3, W4)):
        sagg = sc_aggregate(g.reshape(NCORES * NPAD, LANES), src2, dst2)
        g = _tc_mid(sagg, g, dinv, _bcast8(b), W)
    sagg = sc_aggregate(g.reshape(NCORES * NPAD, LANES), src2, dst2)
    out = _tc_final(sagg, g, dinv, _bcast8(b4),
                    fw1, _bcast8(fb1), fw2, _bcast8(fb2), fw3, _bcast8(fb3))
    return out[:N]


# speedup vs baseline: 19.9150x; 1.0071x over previous
import functools

import jax
import jax.numpy as jnp
from jax import lax
from jax.experimental import pallas as pl
from jax.experimental.pallas import tpu as pltpu
from jax.experimental.pallas import tpu_sc as plsc

N = 10000
NPAD = 10240
E = 320000
LANES = 128
EROWS = 2560
EPAD = EROWS * LANES
NCORES = 2
NSUB = 16
CH = 64
CHROWS = EPAD // CH
TCH = CHROWS // NSUB
IB = 32
NBLK = TCH // IB
DROWS = CHROWS // (NCORES * NSUB)
STRIPE = NPAD // NSUB
RB = 1024
GRID = NPAD // RB
D_IN = 128
D_H = 256


@functools.cache
def _sc_kernels():
    mesh = plsc.VectorSubcoreMesh(core_axis_name="c", subcore_axis_name="s",
                                  num_cores=NCORES, num_subcores=NSUB)

    deg = functools.partial(
        pl.kernel,
        out_type=jax.ShapeDtypeStruct((NCORES, NPAD), jnp.float32),
        mesh=mesh,
        scratch_types=[
            pltpu.VMEM((DROWS, CH), jnp.int32),
            pltpu.VMEM((CH,), jnp.float32),
            pltpu.VMEM((STRIPE,), jnp.float32),
            pltpu.VMEM_SHARED((NPAD,), jnp.float32),
            pltpu.SemaphoreType.DMA,
        ],
    )(_sc_degree_body)

    agg = functools.partial(
        pl.kernel,
        out_type=jax.ShapeDtypeStruct((NCORES, NPAD, LANES), jnp.float32),
        mesh=mesh,
        scratch_types=[
            pltpu.VMEM((IB, CH), jnp.int32),
            pltpu.VMEM((IB, CH), jnp.int32),
            pltpu.VMEM((IB, CH), jnp.int32),
            pltpu.VMEM((IB, CH), jnp.int32),
            pltpu.VMEM((CH, LANES), jnp.float32),
            pltpu.VMEM((CH, LANES), jnp.float32),
            pltpu.VMEM((CH, LANES), jnp.float32),
            pltpu.VMEM((CH, LANES), jnp.float32),
            pltpu.VMEM_SHARED((NPAD, LANES), jnp.float32),
            pltpu.SemaphoreType.DMA, pltpu.SemaphoreType.DMA,
            pltpu.SemaphoreType.DMA, pltpu.SemaphoreType.DMA,
            pltpu.SemaphoreType.DMA, pltpu.SemaphoreType.DMA,
            pltpu.SemaphoreType.DMA, pltpu.SemaphoreType.DMA,
            pltpu.SemaphoreType.DMA, pltpu.SemaphoreType.DMA,
        ],
    )(_sc_aggregate_body)

    return deg, agg


def _sc_degree_body(dst_hbm, out_hbm, dstb, ones, stage, acc, sem):
    c = lax.axis_index("c")
    s = lax.axis_index("s")
    w = s * NCORES + c

    for j in range(CH // 16):
        ones[pl.ds(j * 16, 16)] = jnp.ones((16,), jnp.float32)

    iv = jnp.where(c == 0, 1.0, 0.0).astype(jnp.float32)

    def _fill(i, carry):
        stage[pl.ds(i * 16, 16)] = jnp.broadcast_to(iv, (16,))
        return carry

    lax.fori_loop(0, STRIPE // 16, _fill, 0)
    pltpu.sync_copy(stage, acc.at[pl.ds(s * STRIPE, STRIPE)])
    pltpu.sync_copy(dst_hbm.at[pl.ds(w * DROWS, DROWS)], dstb)
    plsc.subcore_barrier()

    def _grp(gi, carry):
        for u in range(8):
            pltpu.async_copy(ones, acc.at[dstb.at[gi * 8 + u]], sem, add=True)
        for u in range(8):
            pltpu.make_async_copy(ones, acc.at[dstb.at[gi * 8 + u]],
                                  sem).wait()
        return carry

    lax.fori_loop(0, DROWS // 8, _grp, 0)
    plsc.subcore_barrier()

    pltpu.sync_copy(acc.at[pl.ds(s * STRIPE, STRIPE)], stage)
    pltpu.sync_copy(stage, out_hbm.at[c, pl.ds(s * STRIPE, STRIPE)])


def _sc_aggregate_body(g_hbm, src_hbm, dst_hbm, out_hbm,
                       srcb0, dstb0, srcb1, dstb1, b0, b1, b2, b3, acc,
                       sg0, sg1, sg2, sg3, ss0, ss1, ss2, ss3, si0, si1):
    c = lax.axis_index("c")
    s = lax.axis_index("s")
    bufs = (b0, b1, b2, b3)
    sgs = (sg0, sg1, sg2, sg3)
    sss = (ss0, ss1, ss2, ss3)

    def _zrow(i, carry):
        for j in range(LANES // 16):
            b0[i, pl.ds(j * 16, 16)] = jnp.zeros((16,), jnp.float32)
        return carry

    lax.fori_loop(0, CH, _zrow, 0)
    for k in range(STRIPE // CH):
        pltpu.sync_copy(b0, acc.at[pl.ds(s * STRIPE + k * CH, CH)])
    plsc.subcore_barrier()

    def _load_idx(base, sb, db, sem):
        pltpu.async_copy(src_hbm.at[c, pl.ds(base, IB)], sb, sem)
        pltpu.async_copy(dst_hbm.at[pl.ds(base, IB)], db, sem)

    def _wait_idx(base, sb, db, sem):
        pltpu.make_async_copy(src_hbm.at[c, pl.ds(base, IB)], sb, sem).wait()
        pltpu.make_async_copy(dst_hbm.at[pl.ds(base, IB)], db, sem).wait()

    def _process(srcb, dstb):
        pltpu.async_copy(g_hbm.at[srcb.at[0]], b0, sg0)
        pltpu.async_copy(g_hbm.at[srcb.at[1]], b1, sg1)

        def _quad(q, carry2):
            for u in range(4):
                j = 4 * q + u
                v = (u + 2) % 4
                if u < 2:
                    @pl.when(q >= 1)
                    def _():
                        pltpu.make_async_copy(
                            bufs[v], acc.at[dstb.at[j - 2]], sss[v]).wait()
                    pltpu.async_copy(g_hbm.at[srcb.at[j + 2]], bufs[v], sgs[v])
                else:
                    pltpu.make_async_copy(
                        bufs[v], acc.at[dstb.at[j - 2]], sss[v]).wait()

                    @pl.when(q < IB // 4 - 1)
                    def _():
                        pltpu.async_copy(g_hbm.at[srcb.at[j + 2]],
                                         bufs[v], sgs[v])
                pltpu.make_async_copy(g_hbm.at[srcb.at[j]],
                                      bufs[u], sgs[u]).wait()
                pltpu.async_copy(bufs[u], acc.at[dstb.at[j]], sss[u], add=True)
            return carry2

        lax.fori_loop(0, IB // 4, _quad, 0)
        pltpu.make_async_copy(b2, acc.at[dstb.at[IB - 2]], ss2).wait()
        pltpu.make_async_copy(b3, acc.at[dstb.at[IB - 1]], ss3).wait()

    tbase = s * TCH
    _load_idx(tbase, srcb0, dstb0, si0)

    def _bb(t, carry):
        base0 = tbase + 2 * t * IB
        _wait_idx(base0, srcb0, dstb0, si0)
        _load_idx(base0 + IB, srcb1, dstb1, si1)
        _process(srcb0, dstb0)
        _wait_idx(base0 + IB, srcb1, dstb1, si1)

        @pl.when(t < NBLK // 2 - 1)
        def _():
            _load_idx(base0 + 2 * IB, srcb0, dstb0, si0)

        _process(srcb1, dstb1)
        return carry

    lax.fori_loop(0, NBLK // 2, _bb, 0)
    plsc.subcore_barrier()

    for k in range(STRIPE // CH):
        base = s * STRIPE + k * CH
        pltpu.sync_copy(acc.at[pl.ds(base, CH)], b0)
        pltpu.sync_copy(b0, out_hbm.at[c, pl.ds(base, CH)])


def _dinv_col(deg_ref):
    d = deg_ref[0] + deg_ref[1]
    dv = lax.rsqrt(d)
    t = dv.T
    return jnp.concatenate([t[:, k:k + 1] for k in range(8)], axis=0)


def _tc_first_body(deg_ref, x_ref, w_ref, g_ref):
    col = _dinv_col(deg_ref)
    h = jnp.dot(x_ref[...], w_ref[...], preferred_element_type=jnp.float32)
    g = h * col
    g_ref[0] = g[:, :LANES]
    g_ref[1] = g[:, LANES:]


def _tc_mid_body(s_ref, gp_ref, deg_ref, b_ref, w_ref, g_ref):
    col = _dinv_col(deg_ref)
    zl = jnp.maximum((s_ref[0] + gp_ref[0]) * col + b_ref[0:1, :LANES], 0.0)
    zr = jnp.maximum((s_ref[1] + gp_ref[1]) * col + b_ref[0:1, LANES:], 0.0)
    h = (jnp.dot(zl, w_ref[:LANES, :], preferred_element_type=jnp.float32)
         + jnp.dot(zr, w_ref[LANES:, :], preferred_element_type=jnp.float32))
    g = h * col
    g_ref[0] = g[:, :LANES]
    g_ref[1] = g[:, LANES:]


def _tc_final_body(s_ref, gp_ref, deg_ref, b_ref,
                   fw1_ref, fb1_ref, fw2_ref, fb2_ref, fw3_ref, fb3_ref,
                   o_ref):
    col = _dinv_col(deg_ref)
    zl = jnp.maximum((s_ref[0] + gp_ref[0]) * col + b_ref[0:1, :LANES], 0.0)
    zr = jnp.maximum((s_ref[1] + gp_ref[1]) * col + b_ref[0:1, LANES:], 0.0)
    z = jnp.concatenate([zl, zr], axis=1)
    h1 = jnp.maximum(
        jnp.dot(z, fw1_ref[...], preferred_element_type=jnp.float32)
        + fb1_ref[0:1, :], 0.0)
    h2 = jnp.maximum(
        jnp.dot(h1, fw2_ref[...], preferred_element_type=jnp.float32)
        + fb2_ref[0:1, :], 0.0)
    o_ref[...] = (jnp.dot(h2, fw3_ref[...], preferred_element_type=jnp.float32)
                  + fb3_ref[0:1, :])


_f32 = jnp.float32
_gspec = pl.BlockSpec((NCORES, RB, LANES), lambda i: (0, i, 0))
_nspec = pl.BlockSpec((RB, LANES), lambda i: (i, 0))
_gshape = jax.ShapeDtypeStruct((NCORES, NPAD, LANES), _f32)

_dspec = pl.BlockSpec((NCORES, 8, LANES), lambda i: (0, i, 0))
_tc_first_specs = [_dspec, _nspec,
                   pl.BlockSpec((D_IN, D_H), lambda i: (0, 0))]
_tc_first_outspecs = _gspec
_tc_first_outshape = _gshape
_tc_mid_specs = [_gspec, _gspec, _dspec,
                 pl.BlockSpec((8, D_H), lambda i: (0, 0)),
                 pl.BlockSpec((D_H, D_H), lambda i: (0, 0))]
_tc_final_specs = [_gspec, _gspec, _dspec,
                   pl.BlockSpec((8, D_H), lambda i: (0, 0)),
                   pl.BlockSpec((D_H, D_H), lambda i: (0, 0)),
                   pl.BlockSpec((8, D_H), lambda i: (0, 0)),
                   pl.BlockSpec((D_H, LANES), lambda i: (0, 0)),
                   pl.BlockSpec((8, LANES), lambda i: (0, 0)),
                   pl.BlockSpec((LANES, LANES), lambda i: (0, 0)),
                   pl.BlockSpec((8, LANES), lambda i: (0, 0))]
_tc_final_outshape = jax.ShapeDtypeStruct((N, LANES), _f32)

_tc_first = pl.pallas_call(
    _tc_first_body, grid=(GRID,), in_specs=_tc_first_specs,
    out_specs=_tc_first_outspecs, out_shape=_tc_first_outshape)

_tc_mid = pl.pallas_call(
    _tc_mid_body, grid=(GRID,), in_specs=_tc_mid_specs,
    out_specs=_gspec, out_shape=_gshape)

_tc_final = pl.pallas_call(
    _tc_final_body, grid=(GRID,), in_specs=_tc_final_specs,
    out_specs=_nspec, out_shape=_tc_final_outshape)


def _bcast8(b):
    return jnp.broadcast_to(b[None, :], (8, b.shape[0]))


def kernel(x, edge_index, W1, b1, W2, b2, W3, b3, W4, b4,
           fw1, fb1, fw2, fb2, fw3, fb3):
    src = edge_index[0].astype(jnp.int32)
    dst = edge_index[1].astype(jnp.int32)
    pad = EPAD - E
    pi = jnp.arange(pad, dtype=jnp.int32)
    srcp = jnp.concatenate([src, pi % N])
    dstp = jnp.concatenate([dst, N + pi % (NPAD - N)])
    src2 = jnp.stack([srcp, srcp + NPAD]).reshape(NCORES, CHROWS, CH)
    dst2 = dstp.reshape(CHROWS, CH)

    sc_degree, sc_aggregate = _sc_kernels()
    deg = sc_degree(dst2)
    deg2d = deg.reshape(NCORES, NPAD // LANES, LANES)

    g = _tc_first(deg2d, x, W1)
    for b, W in ((b1, W2), (b2, W3), (b3, W4)):
        sagg = sc_aggregate(g.reshape(NCORES * NPAD, LANES), src2, dst2)
        g = _tc_mid(sagg, g, deg2d, _bcast8(b), W)
    sagg = sc_aggregate(g.reshape(NCORES * NPAD, LANES), src2, dst2)
    return _tc_final(sagg, g, deg2d, _bcast8(b4),
                     fw1, _bcast8(fb1), fw2, _bcast8(fb2), fw3, _bcast8(fb3))


# speedup vs baseline: 20.2785x; 1.0182x over previous
import functools

import jax
import jax.numpy as jnp
from jax import lax
from jax.experimental import pallas as pl
from jax.experimental.pallas import tpu as pltpu
from jax.experimental.pallas import tpu_sc as plsc

N = 10000
NPAD = 10240
E = 320000
LANES = 128
EROWS = 2560
EPAD = EROWS * LANES
NCORES = 2
NSUB = 16
CH = 64
CHROWS = EPAD // CH
TCH = CHROWS // NSUB
IB = 32
NBLK = TCH // IB
DROWS = CHROWS // (NCORES * NSUB)
STRIPE = NPAD // NSUB
RB = 1024
GRID = NPAD // RB
D_IN = 128
D_H = 256


@functools.cache
def _sc_kernels():
    mesh = plsc.VectorSubcoreMesh(core_axis_name="c", subcore_axis_name="s",
                                  num_cores=NCORES, num_subcores=NSUB)

    deg = functools.partial(
        pl.kernel,
        out_type=jax.ShapeDtypeStruct((NCORES, NPAD), jnp.float32),
        mesh=mesh,
        scratch_types=[
            pltpu.VMEM((DROWS, CH), jnp.int32),
            pltpu.VMEM((CH,), jnp.float32),
            pltpu.VMEM((STRIPE,), jnp.float32),
            pltpu.VMEM_SHARED((NPAD,), jnp.float32),
            pltpu.SemaphoreType.DMA,
        ],
    )(_sc_degree_body)

    agg = functools.partial(
        pl.kernel,
        out_type=jax.ShapeDtypeStruct((NCORES, NPAD, LANES), jnp.float32),
        mesh=mesh,
        scratch_types=[
            pltpu.VMEM((IB, CH), jnp.int32),
            pltpu.VMEM((IB, CH), jnp.int32),
            pltpu.VMEM((IB, CH), jnp.int32),
            pltpu.VMEM((IB, CH), jnp.int32),
            pltpu.VMEM((CH, LANES), jnp.float32),
            pltpu.VMEM((CH, LANES), jnp.float32),
            pltpu.VMEM((CH, LANES), jnp.float32),
            pltpu.VMEM((CH, LANES), jnp.float32),
            pltpu.VMEM_SHARED((NPAD, LANES), jnp.float32),
            pltpu.SemaphoreType.DMA, pltpu.SemaphoreType.DMA,
            pltpu.SemaphoreType.DMA, pltpu.SemaphoreType.DMA,
            pltpu.SemaphoreType.DMA, pltpu.SemaphoreType.DMA,
            pltpu.SemaphoreType.DMA, pltpu.SemaphoreType.DMA,
            pltpu.SemaphoreType.DMA, pltpu.SemaphoreType.DMA,
        ],
    )(_sc_aggregate_body)

    return deg, agg


def _sc_degree_body(dst_hbm, out_hbm, dstb, ones, stage, acc, sem):
    c = lax.axis_index("c")
    s = lax.axis_index("s")
    w = s * NCORES + c

    for j in range(CH // 16):
        ones[pl.ds(j * 16, 16)] = jnp.ones((16,), jnp.float32)

    iv = jnp.where(c == 0, 1.0, 0.0).astype(jnp.float32)

    def _fill(i, carry):
        stage[pl.ds(i * 16, 16)] = jnp.broadcast_to(iv, (16,))
        return carry

    lax.fori_loop(0, STRIPE // 16, _fill, 0)
    pltpu.sync_copy(stage, acc.at[pl.ds(s * STRIPE, STRIPE)])
    pltpu.sync_copy(dst_hbm.at[pl.ds(w * DROWS, DROWS)], dstb)
    plsc.subcore_barrier()

    def _grp(gi, carry):
        for u in range(8):
            pltpu.async_copy(ones, acc.at[dstb.at[gi * 8 + u]], sem, add=True)
        for u in range(8):
            pltpu.make_async_copy(ones, acc.at[dstb.at[gi * 8 + u]],
                                  sem).wait()
        return carry

    lax.fori_loop(0, DROWS // 8, _grp, 0)
    plsc.subcore_barrier()

    pltpu.sync_copy(acc.at[pl.ds(s * STRIPE, STRIPE)], stage)
    pltpu.sync_copy(stage, out_hbm.at[c, pl.ds(s * STRIPE, STRIPE)])


def _sc_aggregate_body(g_hbm, src_hbm, dst_hbm, out_hbm,
                       srcb0, dstb0, srcb1, dstb1, b0, b1, b2, b3, acc,
                       sg0, sg1, sg2, sg3, ss0, ss1, ss2, ss3, si0, si1):
    c = lax.axis_index("c")
    s = lax.axis_index("s")
    bufs = (b0, b1, b2, b3)
    sgs = (sg0, sg1, sg2, sg3)
    sss = (ss0, ss1, ss2, ss3)

    def _load_idx(base, sb, db, sem):
        pltpu.async_copy(src_hbm.at[c, pl.ds(base, IB)], sb, sem)
        pltpu.async_copy(dst_hbm.at[pl.ds(base, IB)], db, sem)

    def _wait_idx(base, sb, db, sem):
        pltpu.make_async_copy(src_hbm.at[c, pl.ds(base, IB)], sb, sem).wait()
        pltpu.make_async_copy(dst_hbm.at[pl.ds(base, IB)], db, sem).wait()

    tbase = s * TCH
    _load_idx(tbase, srcb0, dstb0, si0)

    def _zrow(i, carry):
        for j in range(LANES // 16):
            b0[i, pl.ds(j * 16, 16)] = jnp.zeros((16,), jnp.float32)
        return carry

    lax.fori_loop(0, CH, _zrow, 0)
    for k in range(STRIPE // CH):
        pltpu.async_copy(b0, acc.at[pl.ds(s * STRIPE + k * CH, CH)], ss0)
    for k in range(STRIPE // CH):
        pltpu.make_async_copy(b0, acc.at[pl.ds(s * STRIPE + k * CH, CH)],
                              ss0).wait()
    plsc.subcore_barrier()

    def _process(srcb, dstb):
        pltpu.async_copy(g_hbm.at[srcb.at[0]], b0, sg0)
        pltpu.async_copy(g_hbm.at[srcb.at[1]], b1, sg1)

        def _quad(q, carry2):
            for u in range(4):
                j = 4 * q + u
                v = (u + 2) % 4
                if u < 2:
                    @pl.when(q >= 1)
                    def _():
                        pltpu.make_async_copy(
                            bufs[v], acc.at[dstb.at[j - 2]], sss[v]).wait()
                    pltpu.async_copy(g_hbm.at[srcb.at[j + 2]], bufs[v], sgs[v])
                else:
                    pltpu.make_async_copy(
                        bufs[v], acc.at[dstb.at[j - 2]], sss[v]).wait()

                    @pl.when(q < IB // 4 - 1)
                    def _():
                        pltpu.async_copy(g_hbm.at[srcb.at[j + 2]],
                                         bufs[v], sgs[v])
                pltpu.make_async_copy(g_hbm.at[srcb.at[j]],
                                      bufs[u], sgs[u]).wait()
                pltpu.async_copy(bufs[u], acc.at[dstb.at[j]], sss[u], add=True)
            return carry2

        lax.fori_loop(0, IB // 4, _quad, 0)
        pltpu.make_async_copy(b2, acc.at[dstb.at[IB - 2]], ss2).wait()
        pltpu.make_async_copy(b3, acc.at[dstb.at[IB - 1]], ss3).wait()

    def _bb(t, carry):
        base0 = tbase + 2 * t * IB
        _wait_idx(base0, srcb0, dstb0, si0)
        _load_idx(base0 + IB, srcb1, dstb1, si1)
        _process(srcb0, dstb0)
        _wait_idx(base0 + IB, srcb1, dstb1, si1)

        @pl.when(t < NBLK // 2 - 1)
        def _():
            _load_idx(base0 + 2 * IB, srcb0, dstb0, si0)

        _process(srcb1, dstb1)
        return carry

    lax.fori_loop(0, NBLK // 2, _bb, 0)
    plsc.subcore_barrier()

    nwb = STRIPE // CH
    for k in range(nwb):
        bb, sem = (b0, ss0) if k % 2 == 0 else (b1, ss1)
        base = s * STRIPE + k * CH
        if k >= 2:
            pbase = s * STRIPE + (k - 2) * CH
            pltpu.make_async_copy(bb, out_hbm.at[c, pl.ds(pbase, CH)],
                                  sem).wait()
        pltpu.sync_copy(acc.at[pl.ds(base, CH)], bb)
        pltpu.async_copy(bb, out_hbm.at[c, pl.ds(base, CH)], sem)
    for k in range(nwb - 2, nwb):
        bb, sem = (b0, ss0) if k % 2 == 0 else (b1, ss1)
        base = s * STRIPE + k * CH
        pltpu.make_async_copy(bb, out_hbm.at[c, pl.ds(base, CH)], sem).wait()


def _dinv_col(deg_ref):
    d = deg_ref[0] + deg_ref[1]
    dv = lax.rsqrt(d)
    t = dv.T
    return jnp.concatenate([t[:, k:k + 1] for k in range(8)], axis=0)


def _tc_first_body(deg_ref, x_ref, w_ref, g_ref):
    col = _dinv_col(deg_ref)
    h = jnp.dot(x_ref[...], w_ref[...], preferred_element_type=jnp.float32)
    g = h * col
    g_ref[0] = g[:, :LANES]
    g_ref[1] = g[:, LANES:]


def _tc_mid_body(s_ref, gp_ref, deg_ref, b_ref, w_ref, g_ref):
    col = _dinv_col(deg_ref)
    zl = jnp.maximum((s_ref[0] + gp_ref[0]) * col + b_ref[0:1, :LANES], 0.0)
    zr = jnp.maximum((s_ref[1] + gp_ref[1]) * col + b_ref[0:1, LANES:], 0.0)
    h = (jnp.dot(zl, w_ref[:LANES, :], preferred_element_type=jnp.float32)
         + jnp.dot(zr, w_ref[LANES:, :], preferred_element_type=jnp.float32))
    g = h * col
    g_ref[0] = g[:, :LANES]
    g_ref[1] = g[:, LANES:]


def _tc_final_body(s_ref, gp_ref, deg_ref, b_ref,
                   fw1_ref, fb1_ref, fw2_ref, fb2_ref, fw3_ref, fb3_ref,
                   o_ref):
    col = _dinv_col(deg_ref)
    zl = jnp.maximum((s_ref[0] + gp_ref[0]) * col + b_ref[0:1, :LANES], 0.0)
    zr = jnp.maximum((s_ref[1] + gp_ref[1]) * col + b_ref[0:1, LANES:], 0.0)
    z = jnp.concatenate([zl, zr], axis=1)
    h1 = jnp.maximum(
        jnp.dot(z, fw1_ref[...], preferred_element_type=jnp.float32)
        + fb1_ref[0:1, :], 0.0)
    h2 = jnp.maximum(
        jnp.dot(h1, fw2_ref[...], preferred_element_type=jnp.float32)
        + fb2_ref[0:1, :], 0.0)
    o_ref[...] = (jnp.dot(h2, fw3_ref[...], preferred_element_type=jnp.float32)
                  + fb3_ref[0:1, :])


_f32 = jnp.float32
_gspec = pl.BlockSpec((NCORES, RB, LANES), lambda i: (0, i, 0))
_nspec = pl.BlockSpec((RB, LANES), lambda i: (i, 0))
_gshape = jax.ShapeDtypeStruct((NCORES, NPAD, LANES), _f32)

_dspec = pl.BlockSpec((NCORES, 8, LANES), lambda i: (0, i, 0))
_tc_first_specs = [_dspec, _nspec,
                   pl.BlockSpec((D_IN, D_H), lambda i: (0, 0))]
_tc_first_outspecs = _gspec
_tc_first_outshape = _gshape
_tc_mid_specs = [_gspec, _gspec, _dspec,
                 pl.BlockSpec((8, D_H), lambda i: (0, 0)),
                 pl.BlockSpec((D_H, D_H), lambda i: (0, 0))]
_tc_final_specs = [_gspec, _gspec, _dspec,
                   pl.BlockSpec((8, D_H), lambda i: (0, 0)),
                   pl.BlockSpec((D_H, D_H), lambda i: (0, 0)),
                   pl.BlockSpec((8, D_H), lambda i: (0, 0)),
                   pl.BlockSpec((D_H, LANES), lambda i: (0, 0)),
                   pl.BlockSpec((8, LANES), lambda i: (0, 0)),
                   pl.BlockSpec((LANES, LANES), lambda i: (0, 0)),
                   pl.BlockSpec((8, LANES), lambda i: (0, 0))]
_tc_final_outshape = jax.ShapeDtypeStruct((N, LANES), _f32)

_tc_first = pl.pallas_call(
    _tc_first_body, grid=(GRID,), in_specs=_tc_first_specs,
    out_specs=_tc_first_outspecs, out_shape=_tc_first_outshape)

_tc_mid = pl.pallas_call(
    _tc_mid_body, grid=(GRID,), in_specs=_tc_mid_specs,
    out_specs=_gspec, out_shape=_gshape)

_tc_final = pl.pallas_call(
    _tc_final_body, grid=(GRID,), in_specs=_tc_final_specs,
    out_specs=_nspec, out_shape=_tc_final_outshape)


def _bcast8(b):
    return jnp.broadcast_to(b[None, :], (8, b.shape[0]))


def kernel(x, edge_index, W1, b1, W2, b2, W3, b3, W4, b4,
           fw1, fb1, fw2, fb2, fw3, fb3):
    src = edge_index[0].astype(jnp.int32)
    dst = edge_index[1].astype(jnp.int32)
    pad = EPAD - E
    pi = jnp.arange(pad, dtype=jnp.int32)
    srcp = jnp.concatenate([src, pi % N])
    dstp = jnp.concatenate([dst, N + pi % (NPAD - N)])
    src2 = jnp.stack([srcp, srcp + NPAD]).reshape(NCORES, CHROWS, CH)
    dst2 = dstp.reshape(CHROWS, CH)

    sc_degree, sc_aggregate = _sc_kernels()
    deg = sc_degree(dst2)
    deg2d = deg.reshape(NCORES, NPAD // LANES, LANES)

    g = _tc_first(deg2d, x, W1)
    for b, W in ((b1, W2), (b2, W3), (b3, W4)):
        sagg = sc_aggregate(g.reshape(NCORES * NPAD, LANES), src2, dst2)
        g = _tc_mid(sagg, g, deg2d, _bcast8(b), W)
    sagg = sc_aggregate(g.reshape(NCORES * NPAD, LANES), src2, dst2)
    return _tc_final(sagg, g, deg2d, _bcast8(b4),
                     fw1, _bcast8(fb1), fw2, _bcast8(fb2), fw3, _bcast8(fb3))


# speedup vs baseline: 20.5872x; 1.0152x over previous
import functools

import jax
import jax.numpy as jnp
from jax import lax
from jax.experimental import pallas as pl
from jax.experimental.pallas import tpu as pltpu
from jax.experimental.pallas import tpu_sc as plsc

N = 10000
NPAD = 10240
E = 320000
LANES = 128
EROWS = 2560
EPAD = EROWS * LANES
NCORES = 2
NSUB = 16
CH = 64
CHROWS = EPAD // CH
TCH = CHROWS // NSUB
IB = 32
NBLK = TCH // IB
DROWS = CHROWS // (NCORES * NSUB)
STRIPE = NPAD // NSUB
RB = 1024
GRID = NPAD // RB
D_IN = 128
D_H = 256


@functools.cache
def _sc_kernels():
    mesh = plsc.VectorSubcoreMesh(core_axis_name="c", subcore_axis_name="s",
                                  num_cores=NCORES, num_subcores=NSUB)

    deg = functools.partial(
        pl.kernel,
        out_type=jax.ShapeDtypeStruct((NCORES, NPAD), jnp.float32),
        mesh=mesh,
        scratch_types=[
            pltpu.VMEM((DROWS, CH), jnp.int32),
            pltpu.VMEM((CH,), jnp.float32),
            pltpu.VMEM((STRIPE,), jnp.float32),
            pltpu.VMEM_SHARED((NPAD,), jnp.float32),
            pltpu.SemaphoreType.DMA,
        ],
    )(_sc_degree_body)

    agg = functools.partial(
        pl.kernel,
        out_type=jax.ShapeDtypeStruct((NCORES, NPAD, LANES), jnp.float32),
        mesh=mesh,
        scratch_types=[
            pltpu.VMEM((IB, CH), jnp.int32),
            pltpu.VMEM((IB, CH), jnp.int32),
            pltpu.VMEM((IB, CH), jnp.int32),
            pltpu.VMEM((IB, CH), jnp.int32),
            pltpu.VMEM((CH, LANES), jnp.float32),
            pltpu.VMEM((CH, LANES), jnp.float32),
            pltpu.VMEM((CH, LANES), jnp.float32),
            pltpu.VMEM((CH, LANES), jnp.float32),
            pltpu.VMEM_SHARED((NPAD, LANES), jnp.float32),
            pltpu.SemaphoreType.DMA, pltpu.SemaphoreType.DMA,
            pltpu.SemaphoreType.DMA, pltpu.SemaphoreType.DMA,
            pltpu.SemaphoreType.DMA, pltpu.SemaphoreType.DMA,
            pltpu.SemaphoreType.DMA, pltpu.SemaphoreType.DMA,
            pltpu.SemaphoreType.DMA, pltpu.SemaphoreType.DMA,
        ],
    )(_sc_aggregate_body)

    return deg, agg


def _sc_degree_body(dst_hbm, out_hbm, dstb, ones, stage, acc, sem):
    c = lax.axis_index("c")
    s = lax.axis_index("s")
    w = s * NCORES + c

    for j in range(CH // 16):
        ones[pl.ds(j * 16, 16)] = jnp.ones((16,), jnp.float32)

    iv = jnp.where(c == 0, 1.0, 0.0).astype(jnp.float32)

    def _fill(i, carry):
        stage[pl.ds(i * 16, 16)] = jnp.broadcast_to(iv, (16,))
        return carry

    lax.fori_loop(0, STRIPE // 16, _fill, 0)
    pltpu.sync_copy(stage, acc.at[pl.ds(s * STRIPE, STRIPE)])
    pltpu.sync_copy(dst_hbm.at[pl.ds(w * DROWS, DROWS)], dstb)
    plsc.subcore_barrier()

    def _grp(gi, carry):
        for u in range(8):
            pltpu.async_copy(ones, acc.at[dstb.at[gi * 8 + u]], sem, add=True)
        for u in range(8):
            pltpu.make_async_copy(ones, acc.at[dstb.at[gi * 8 + u]],
                                  sem).wait()
        return carry

    lax.fori_loop(0, DROWS // 8, _grp, 0)
    plsc.subcore_barrier()

    pltpu.sync_copy(acc.at[pl.ds(s * STRIPE, STRIPE)], stage)
    pltpu.sync_copy(stage, out_hbm.at[c, pl.ds(s * STRIPE, STRIPE)])


def _sc_aggregate_body(g_hbm, src_hbm, dst_hbm, out_hbm,
                       srcb0, dstb0, srcb1, dstb1, b0, b1, b2, b3, acc,
                       sg0, sg1, sg2, sg3, ss0, ss1, ss2, ss3, si0, si1):
    c = lax.axis_index("c")
    s = lax.axis_index("s")
    gc = g_hbm.at[c]
    bufs = (b0, b1, b2, b3)
    sgs = (sg0, sg1, sg2, sg3)
    sss = (ss0, ss1, ss2, ss3)

    def _load_idx(base, sb, db, sem):
        pltpu.async_copy(src_hbm.at[pl.ds(base, IB)], sb, sem)
        pltpu.async_copy(dst_hbm.at[pl.ds(base, IB)], db, sem)

    def _wait_idx(base, sb, db, sem):
        pltpu.make_async_copy(src_hbm.at[pl.ds(base, IB)], sb, sem).wait()
        pltpu.make_async_copy(dst_hbm.at[pl.ds(base, IB)], db, sem).wait()

    tbase = s * TCH
    _load_idx(tbase, srcb0, dstb0, si0)

    def _zrow(i, carry):
        for j in range(LANES // 16):
            b0[i, pl.ds(j * 16, 16)] = jnp.zeros((16,), jnp.float32)
        return carry

    lax.fori_loop(0, CH, _zrow, 0)
    for k in range(STRIPE // CH):
        pltpu.async_copy(b0, acc.at[pl.ds(s * STRIPE + k * CH, CH)], ss0)
    for k in range(STRIPE // CH):
        pltpu.make_async_copy(b0, acc.at[pl.ds(s * STRIPE + k * CH, CH)],
                              ss0).wait()
    plsc.subcore_barrier()

    def _process(srcb, dstb):
        pltpu.async_copy(gc.at[srcb.at[0]], b0, sg0)
        pltpu.async_copy(gc.at[srcb.at[1]], b1, sg1)

        def _quad(q, carry2):
            for u in range(4):
                j = 4 * q + u
                v = (u + 2) % 4
                if u < 2:
                    @pl.when(q >= 1)
                    def _():
                        pltpu.make_async_copy(
                            bufs[v], acc.at[dstb.at[j - 2]], sss[v]).wait()
                    pltpu.async_copy(gc.at[srcb.at[j + 2]], bufs[v], sgs[v])
                else:
                    pltpu.make_async_copy(
                        bufs[v], acc.at[dstb.at[j - 2]], sss[v]).wait()

                    @pl.when(q < IB // 4 - 1)
                    def _():
                        pltpu.async_copy(gc.at[srcb.at[j + 2]],
                                         bufs[v], sgs[v])
                pltpu.make_async_copy(gc.at[srcb.at[j]],
                                      bufs[u], sgs[u]).wait()
                pltpu.async_copy(bufs[u], acc.at[dstb.at[j]], sss[u], add=True)
            return carry2

        lax.fori_loop(0, IB // 4, _quad, 0)
        pltpu.make_async_copy(b2, acc.at[dstb.at[IB - 2]], ss2).wait()
        pltpu.make_async_copy(b3, acc.at[dstb.at[IB - 1]], ss3).wait()

    def _bb(t, carry):
        base0 = tbase + 2 * t * IB
        _wait_idx(base0, srcb0, dstb0, si0)
        _load_idx(base0 + IB, srcb1, dstb1, si1)
        _process(srcb0, dstb0)
        _wait_idx(base0 + IB, srcb1, dstb1, si1)

        @pl.when(t < NBLK // 2 - 1)
        def _():
            _load_idx(base0 + 2 * IB, srcb0, dstb0, si0)

        _process(srcb1, dstb1)
        return carry

    lax.fori_loop(0, NBLK // 2, _bb, 0)
    plsc.subcore_barrier()

    nwb = STRIPE // CH
    for k in range(nwb):
        bb, sem = (b0, ss0) if k % 2 == 0 else (b1, ss1)
        base = s * STRIPE + k * CH
        if k >= 2:
            pbase = s * STRIPE + (k - 2) * CH
            pltpu.make_async_copy(bb, out_hbm.at[c, pl.ds(pbase, CH)],
                                  sem).wait()
        pltpu.sync_copy(acc.at[pl.ds(base, CH)], bb)
        pltpu.async_copy(bb, out_hbm.at[c, pl.ds(base, CH)], sem)
    for k in range(nwb - 2, nwb):
        bb, sem = (b0, ss0) if k % 2 == 0 else (b1, ss1)
        base = s * STRIPE + k * CH
        pltpu.make_async_copy(bb, out_hbm.at[c, pl.ds(base, CH)], sem).wait()


def _dinv_col(deg_ref):
    d = deg_ref[0] + deg_ref[1]
    dv = lax.rsqrt(d)
    t = dv.T
    return jnp.concatenate([t[:, k:k + 1] for k in range(8)], axis=0)


def _tc_first_body(deg_ref, x_ref, w_ref, g_ref):
    col = _dinv_col(deg_ref)
    h = jnp.dot(x_ref[...], w_ref[...], preferred_element_type=jnp.float32)
    g = h * col
    g_ref[0] = g[:, :LANES]
    g_ref[1] = g[:, LANES:]


def _tc_mid_body(s_ref, gp_ref, deg_ref, b_ref, w_ref, g_ref):
    col = _dinv_col(deg_ref)
    zl = jnp.maximum((s_ref[0] + gp_ref[0]) * col + b_ref[0:1, :LANES], 0.0)
    zr = jnp.maximum((s_ref[1] + gp_ref[1]) * col + b_ref[0:1, LANES:], 0.0)
    h = (jnp.dot(zl, w_ref[:LANES, :], preferred_element_type=jnp.float32)
         + jnp.dot(zr, w_ref[LANES:, :], preferred_element_type=jnp.float32))
    g = h * col
    g_ref[0] = g[:, :LANES]
    g_ref[1] = g[:, LANES:]


def _tc_final_body(s_ref, gp_ref, deg_ref, b_ref,
                   fw1_ref, fb1_ref, fw2_ref, fb2_ref, fw3_ref, fb3_ref,
                   o_ref):
    col = _dinv_col(deg_ref)
    zl = jnp.maximum((s_ref[0] + gp_ref[0]) * col + b_ref[0:1, :LANES], 0.0)
    zr = jnp.maximum((s_ref[1] + gp_ref[1]) * col + b_ref[0:1, LANES:], 0.0)
    z = jnp.concatenate([zl, zr], axis=1)
    h1 = jnp.maximum(
        jnp.dot(z, fw1_ref[...], preferred_element_type=jnp.float32)
        + fb1_ref[0:1, :], 0.0)
    h2 = jnp.maximum(
        jnp.dot(h1, fw2_ref[...], preferred_element_type=jnp.float32)
        + fb2_ref[0:1, :], 0.0)
    o_ref[...] = (jnp.dot(h2, fw3_ref[...], preferred_element_type=jnp.float32)
                  + fb3_ref[0:1, :])


_f32 = jnp.float32
_gspec = pl.BlockSpec((NCORES, RB, LANES), lambda i: (0, i, 0))
_nspec = pl.BlockSpec((RB, LANES), lambda i: (i, 0))
_gshape = jax.ShapeDtypeStruct((NCORES, NPAD, LANES), _f32)

_dspec = pl.BlockSpec((NCORES, 8, LANES), lambda i: (0, i, 0))
_tc_first_specs = [_dspec, _nspec,
                   pl.BlockSpec((D_IN, D_H), lambda i: (0, 0))]
_tc_first_outspecs = _gspec
_tc_first_outshape = _gshape
_tc_mid_specs = [_gspec, _gspec, _dspec,
                 pl.BlockSpec((8, D_H), lambda i: (0, 0)),
                 pl.BlockSpec((D_H, D_H), lambda i: (0, 0))]
_tc_final_specs = [_gspec, _gspec, _dspec,
                   pl.BlockSpec((8, D_H), lambda i: (0, 0)),
                   pl.BlockSpec((D_H, D_H), lambda i: (0, 0)),
                   pl.BlockSpec((8, D_H), lambda i: (0, 0)),
                   pl.BlockSpec((D_H, LANES), lambda i: (0, 0)),
                   pl.BlockSpec((8, LANES), lambda i: (0, 0)),
                   pl.BlockSpec((LANES, LANES), lambda i: (0, 0)),
                   pl.BlockSpec((8, LANES), lambda i: (0, 0))]
_tc_final_outshape = jax.ShapeDtypeStruct((N, LANES), _f32)

_tc_first = pl.pallas_call(
    _tc_first_body, grid=(GRID,), in_specs=_tc_first_specs,
    out_specs=_tc_first_outspecs, out_shape=_tc_first_outshape)

_tc_mid = pl.pallas_call(
    _tc_mid_body, grid=(GRID,), in_specs=_tc_mid_specs,
    out_specs=_gspec, out_shape=_gshape)

_tc_final = pl.pallas_call(
    _tc_final_body, grid=(GRID,), in_specs=_tc_final_specs,
    out_specs=_nspec, out_shape=_tc_final_outshape)


def _bcast8(b):
    return jnp.broadcast_to(b[None, :], (8, b.shape[0]))


def kernel(x, edge_index, W1, b1, W2, b2, W3, b3, W4, b4,
           fw1, fb1, fw2, fb2, fw3, fb3):
    src = edge_index[0].astype(jnp.int32)
    dst = edge_index[1].astype(jnp.int32)
    pad = EPAD - E
    pi = jnp.arange(pad, dtype=jnp.int32)
    srcp = jnp.concatenate([src, pi % N])
    dstp = jnp.concatenate([dst, N + pi % (NPAD - N)])
    src2 = srcp.reshape(CHROWS, CH)
    dst2 = dstp.reshape(CHROWS, CH)

    sc_degree, sc_aggregate = _sc_kernels()
    deg = sc_degree(dst2)
    deg2d = deg.reshape(NCORES, NPAD // LANES, LANES)

    g = _tc_first(deg2d, x, W1)
    for b, W in ((b1, W2), (b2, W3), (b3, W4)):
        sagg = sc_aggregate(g, src2, dst2)
        g = _tc_mid(sagg, g, deg2d, _bcast8(b), W)
    sagg = sc_aggregate(g, src2, dst2)
    return _tc_final(sagg, g, deg2d, _bcast8(b4),
                     fw1, _bcast8(fb1), fw2, _bcast8(fb2), fw3, _bcast8(fb3))
